# baseline (speedup 1.0000x reference)
import functools

import jax
import jax.numpy as jnp
from jax import lax
from jax.experimental import pallas as pl
from jax.experimental.pallas import tpu as pltpu
from jax.experimental.pallas import tpu_sc as plsc

NUM_NODES = 100000
NUM_EDGES = 1600000
NC = 2
NS = 16
NW = NC * NS
CHUNK = 128
STAGE = 8
TILE_CHUNKS = 392
OUTER = TILE_CHUNKS // STAGE
TOTAL_CHUNKS = TILE_CHUNKS * NW
EDGES_PAD = TOTAL_CHUNKS * CHUNK
NODES_PAD = 100352
SLICE = NODES_PAD // NS


@functools.lru_cache(maxsize=None)
def _make_sc_pass(feat, do_gather):
  mesh = plsc.VectorSubcoreMesh(core_axis_name="c", subcore_axis_name="s",
                                num_cores=NC, num_subcores=NS)
  frows = SLICE * feat // 128
  scratch = []
  if do_gather:
    scratch.append(pltpu.VMEM((STAGE, CHUNK), jnp.int32))
  scratch += [
      pltpu.VMEM((STAGE, CHUNK), jnp.int32),
      pltpu.VMEM((STAGE, CHUNK, feat), jnp.float32),
      pltpu.VMEM_SHARED((NODES_PAD, feat), jnp.float32),
      pltpu.SemaphoreType.DMA,
      pltpu.SemaphoreType.DMA,
  ]

  def body(*refs):
    if do_gather:
      (table, src2d, dst2d, zeros_blk, out,
       src_v, dst_v, rows_v, acc, sem_g, sem_s) = refs
    else:
      (table, dst2d, zeros_blk, out,
       dst_v, rows_v, acc, sem_g, sem_s) = refs
      src2d = src_v = None
    cid = lax.axis_index("c")
    sid = lax.axis_index("s")
    wid = sid * NC + cid
    pltpu.sync_copy(zeros_blk, acc.at[pl.ds(sid * SLICE, SLICE)])
    if not do_gather:
      for j in range(STAGE):
        pltpu.sync_copy(table, rows_v.at[j])
    plsc.subcore_barrier()

    row0 = wid * TILE_CHUNKS

    def outer(o, carry):
      base = row0 + o * STAGE
      pltpu.sync_copy(dst2d.at[pl.ds(base, STAGE)], dst_v)
      if do_gather:
        pltpu.sync_copy(src2d.at[pl.ds(base, STAGE)], src_v)
        gd = [pltpu.async_copy(table.at[src_v.at[j]], rows_v.at[j], sem_g)
              for j in range(STAGE)]
      sd = []
      for j in range(STAGE):
        if do_gather:
          gd[j].wait()
        sd.append(pltpu.async_copy(rows_v.at[j], acc.at[dst_v.at[j]], sem_s,
                                   add=True))
      for d in sd:
        d.wait()
      return carry

    lax.fori_loop(0, OUTER, outer, 0)
    plsc.subcore_barrier()
    pltpu.sync_copy(acc.at[pl.ds(sid * SLICE, SLICE)],
                    out.at[cid, pl.ds(sid * SLICE, SLICE)])

  return functools.partial(
      pl.kernel,
      out_type=jax.ShapeDtypeStruct((NC, NODES_PAD, feat), jnp.float32),
      mesh=mesh,
      scratch_types=scratch,
      compiler_params=pltpu.CompilerParams(use_tc_tiling_on_sc=False),
  )(body)


GRID = 49
XR = 64
DR = 16
PR = 64


def _tc_prep_body(degp_ref, dis_ref):
  deg = degp_ref[0] + degp_ref[1] + 1.0
  dis_ref[...] = lax.rsqrt(deg)


_tc_prep = pl.pallas_call(
    _tc_prep_body,
    grid=(GRID,),
    in_specs=[pl.BlockSpec((NC, DR, 128), lambda i: (0, i, 0))],
    out_specs=pl.BlockSpec((DR, 128), lambda i: (i, 0)),
    out_shape=jax.ShapeDtypeStruct((NODES_PAD // 128, 128), jnp.float32),
)


def _tc_scale_body(x_ref, dis4_ref, g1_ref):
  g1_ref[...] = x_ref[...] * dis4_ref[...]


_tc_scale = pl.pallas_call(
    _tc_scale_body,
    grid=(GRID,),
    in_specs=[
        pl.BlockSpec((XR, 128), lambda i: (i, 0)),
        pl.BlockSpec((XR, 128), lambda i: (i, 0)),
    ],
    out_specs=pl.BlockSpec((XR, 128), lambda i: (i, 0)),
    out_shape=jax.ShapeDtypeStruct((NODES_PAD * 4 // 128, 128), jnp.float32),
)


def _tc_mid_body(a1p_ref, x_ref, dis4_ref, dis2_ref, w1_ref, b1_ref, w2_ref,
                 g2_ref, sp2_ref):
  dis4 = dis4_ref[...]
  a1 = a1p_ref[0] + a1p_ref[1]
  z = dis4 * a1 + dis4 * dis4 * x_ref[...]
  h = jnp.dot(z, w1_ref[...], preferred_element_type=jnp.float32)
  h = jnp.maximum(h + b1_ref[...], 0.0)
  p = jnp.dot(h, w2_ref[...], preferred_element_type=jnp.float32)
  dis2 = dis2_ref[...]
  g2_ref[...] = dis2 * p
  sp2_ref[...] = dis2 * dis2 * p


_tc_mid = pl.pallas_call(
    _tc_mid_body,
    grid=(GRID,),
    in_specs=[
        pl.BlockSpec((NC, XR, 128), lambda i: (0, i, 0)),
        pl.BlockSpec((XR, 128), lambda i: (i, 0)),
        pl.BlockSpec((XR, 128), lambda i: (i, 0)),
        pl.BlockSpec((PR, 64), lambda i: (i, 0)),
        pl.BlockSpec((128, 2048), lambda i: (0, 0)),
        pl.BlockSpec((1, 2048), lambda i: (0, 0)),
        pl.BlockSpec((2048, 64), lambda i: (0, 0)),
    ],
    out_specs=[
        pl.BlockSpec((PR, 64), lambda i: (i, 0)),
        pl.BlockSpec((PR, 64), lambda i: (i, 0)),
    ],
    out_shape=[
        jax.ShapeDtypeStruct((NODES_PAD * 2 // 64, 64), jnp.float32),
        jax.ShapeDtypeStruct((NODES_PAD * 2 // 64, 64), jnp.float32),
    ],
)


def _tc_final_body(a2p_ref, sp2_ref, dis2_ref, b2_ref, out_ref):
  out_ref[...] = (dis2_ref[...] * (a2p_ref[0] + a2p_ref[1])
                  + sp2_ref[...] + b2_ref[...])


_tc_final = pl.pallas_call(
    _tc_final_body,
    grid=(GRID,),
    in_specs=[
        pl.BlockSpec((NC, PR, 64), lambda i: (0, i, 0)),
        pl.BlockSpec((PR, 64), lambda i: (i, 0)),
        pl.BlockSpec((PR, 64), lambda i: (i, 0)),
        pl.BlockSpec((1, 64), lambda i: (0, 0)),
    ],
    out_specs=pl.BlockSpec((PR, 64), lambda i: (i, 0)),
    out_shape=jax.ShapeDtypeStruct((NODES_PAD * 2 // 64, 64), jnp.float32),
)


def kernel(x, edge_index, W1, b1, W2, b2):
  pad = EDGES_PAD - NUM_EDGES
  src2d = jnp.pad(edge_index[0], (0, pad)).reshape(TOTAL_CHUNKS, CHUNK)
  dst2d = jnp.pad(edge_index[1], (0, pad),
                  constant_values=NUM_NODES).reshape(TOTAL_CHUNKS, CHUNK)
  x_flat = jnp.pad(x, ((0, NODES_PAD - NUM_NODES), (0, 0))).reshape(
      NODES_PAD * 4 // 128, 128)
  ones_rows = jnp.ones((CHUNK, 1), jnp.float32)
  eye32 = jnp.eye(32, dtype=jnp.float32)
  bw1 = jnp.kron(eye32, W1)
  bw2 = jnp.kron(eye32, W2)
  bb1 = jnp.tile(b1, 32).reshape(1, 2048)
  bb2 = jnp.tile(b2, 32).reshape(1, 64)
  z1 = jnp.zeros((SLICE, 1), jnp.float32)
  z4 = jnp.zeros((SLICE, 4), jnp.float32)
  z2 = jnp.zeros((SLICE, 2), jnp.float32)

  degp = _make_sc_pass(1, False)(ones_rows, dst2d, z1)
  dis = _tc_prep(degp.reshape(NC, NODES_PAD // 128, 128))
  dis_f = dis.reshape(-1)
  dis4 = jnp.repeat(dis_f, 4).reshape(NODES_PAD * 4 // 128, 128)
  dis2 = jnp.repeat(dis_f, 2).reshape(NODES_PAD * 2 // 64, 64)
  g1 = _tc_scale(x_flat, dis4)
  a1p = _make_sc_pass(4, True)(g1.reshape(NODES_PAD, 4), src2d, dst2d, z4)
  g2, sp2 = _tc_mid(a1p.reshape(NC, NODES_PAD * 4 // 128, 128), x_flat,
                    dis4, dis2, bw1, bb1, bw2)
  a2p = _make_sc_pass(2, True)(g2.reshape(NODES_PAD, 2), src2d, dst2d, z2)
  out = _tc_final(a2p.reshape(NC, NODES_PAD * 2 // 64, 64), sp2, dis2, bb2)
  return out.reshape(-1)[:NUM_NODES * 2].reshape(NUM_NODES, 2)
